# baseline (speedup 1.0000x reference)
import functools

import jax
import jax.numpy as jnp
from jax import lax
from jax.experimental import pallas as pl
from jax.experimental.pallas import tpu as pltpu, tpu_sc as plsc

_N = 10000
_D = 128
_E = 320000
_EPS = 1e-5

_NP = 10240
_EPAD = 327680
_EPT = _EPAD // 32
_CH = 128
_NCHUNK = 2 * _EPT // _CH
_BC = 8
_CH0 = 112
_CH1 = _NCHUNK - _CH0
_RPT = _NP // 16
_CHC = 128
_NCHUNKC = _EPT // _CHC

_SC_CACHE = {}


def _sc_kernels():
    if "k" in _SC_CACHE:
        return _SC_CACHE["k"]
    mesh = plsc.VectorSubcoreMesh(core_axis_name="c", subcore_axis_name="s")

    @functools.partial(
        pl.kernel,
        mesh=mesh,
        out_type=jax.ShapeDtypeStruct((2, _NP), jnp.float32),
        scratch_types=[
            pltpu.VMEM((_EPT,), jnp.int32),
            pltpu.VMEM((_CHC,), jnp.int32),
            pltpu.VMEM((_CHC,), jnp.float32),
            pltpu.VMEM((_RPT,), jnp.float32),
            pltpu.VMEM_SHARED((_NP,), jnp.float32),
        ],
    )
    def sc_counts(dst_hbm, out_hbm, dst_1d, dst_c, ones_v, zero_v, acc_sh):
        cid = lax.axis_index("c")
        sid = lax.axis_index("s")
        wid = sid * 2 + cid
        for k in range(_CHC // 16):
            ones_v[pl.ds(k * 16, 16)] = jnp.ones((16,), jnp.float32)

        def zbody(k, carry):
            zero_v[pl.ds(k * 16, 16)] = jnp.zeros((16,), jnp.float32)
            return carry

        lax.fori_loop(0, _RPT // 16, zbody, 0)
        pltpu.sync_copy(zero_v, acc_sh.at[pl.ds(sid * _RPT, _RPT)])
        pltpu.sync_copy(dst_hbm.at[pl.ds(wid * _EPT, _EPT)], dst_1d)
        plsc.subcore_barrier()

        def body(j, carry):
            for k in range(_CHC // 16):
                dst_c[pl.ds(k * 16, 16)] = dst_1d[pl.ds(j * _CHC + k * 16, 16)]
            pltpu.sync_copy(ones_v, acc_sh.at[dst_c], add=True)
            return carry

        lax.fori_loop(0, _NCHUNKC, body, 0)
        plsc.subcore_barrier()
        pltpu.sync_copy(acc_sh.at[pl.ds(sid * _RPT, _RPT)],
                        out_hbm.at[cid, pl.ds(sid * _RPT, _RPT)])

    @functools.partial(
        pl.kernel,
        mesh=mesh,
        out_type=jax.ShapeDtypeStruct((2, _NP, _D), jnp.float32),
        scratch_types=(
            [pltpu.VMEM((_BC * _CH,), jnp.int32) for _ in range(4)]
            + [pltpu.VMEM((_CH,), jnp.int32) for _ in range(2)]
            + [pltpu.VMEM((_CH, _D), jnp.float32) for _ in range(2)]
            + [pltpu.VMEM_SHARED((_NP, _D), jnp.float32)]
            + [pltpu.SemaphoreType.DMA for _ in range(6)]
        ),
    )
    def sc_spmm(xp_hbm, src_hbm, dst_hbm, out_hbm, *scratch):
        sblk = scratch[0:2]
        dblk = scratch[2:4]
        dst_v = scratch[4:6]
        rows = scratch[6:8]
        acc_sh = scratch[8]
        sems = scratch[9:11]
        sem_s = scratch[11:13]
        sem_d = scratch[13:15]
        cid = lax.axis_index("c")
        sid = lax.axis_index("s")

        def zbody(r, carry):
            for k in range(_D // 16):
                rows[0][r, pl.ds(k * 16, 16)] = jnp.zeros((16,), jnp.float32)
            return carry

        lax.fori_loop(0, _CH, zbody, 0)
        for k in range(_RPT // _CH):
            pltpu.sync_copy(rows[0],
                            acc_sh.at[pl.ds(sid * _RPT + k * _CH, _CH), :])
        plsc.subcore_barrier()

        def run_edges(base_chunk, nblocks):
            def load_blk(b, q):
                base = (base_chunk + b * _BC) * _CH
                pltpu.async_copy(src_hbm.at[pl.ds(base, _BC * _CH)],
                                 sblk[q], sem_s[q])
                pltpu.async_copy(dst_hbm.at[pl.ds(base, _BC * _CH)],
                                 dblk[q], sem_d[q])

            def wait_blk(q):
                pltpu.make_async_copy(src_hbm.at[pl.ds(0, _BC * _CH)],
                                      sblk[q], sem_s[q]).wait()
                pltpu.make_async_copy(dst_hbm.at[pl.ds(0, _BC * _CH)],
                                      dblk[q], sem_d[q]).wait()

            def do_block(q):
                def fetch(i, p):
                    for k in range(_CH // 16):
                        dst_v[p][pl.ds(k * 16, 16)] = (
                            dblk[q][pl.ds(i * _CH + k * 16, 16)])
                    pltpu.async_copy(
                        xp_hbm.at[sblk[q].at[pl.ds(i * _CH, _CH)]],
                        rows[p], sems[p])

                def drain(p):
                    pltpu.make_async_copy(
                        xp_hbm.at[sblk[q].at[pl.ds(0, _CH)]],
                        rows[p], sems[p]).wait()
                    pltpu.sync_copy(rows[p], acc_sh.at[dst_v[p]], add=True)

                fetch(0, 0)
                for m in range(_BC // 2):
                    fetch(2 * m + 1, 1)
                    drain(0)
                    if m < _BC // 2 - 1:
                        fetch(2 * m + 2, 0)
                    drain(1)

            load_blk(0, 0)

            def body(m, carry):
                load_blk(2 * m + 1, 1)
                wait_blk(0)
                do_block(0)

                @pl.when(m < nblocks // 2 - 1)
                def _():
                    load_blk(2 * m + 2, 0)

                wait_blk(1)
                do_block(1)
                return carry

            lax.fori_loop(0, nblocks // 2, body, 0)

        @pl.when(cid == 0)
        def _():
            run_edges(sid * _NCHUNK, _CH0 // _BC)

        @pl.when(cid == 1)
        def _():
            run_edges(sid * _NCHUNK + _CH0, _CH1 // _BC)
        plsc.subcore_barrier()
        pltpu.sync_copy(acc_sh.at[pl.ds(sid * _RPT, _RPT), :],
                        out_hbm.at[cid, pl.ds(sid * _RPT, _RPT), :])

    _SC_CACHE["k"] = (sc_counts, sc_spmm)
    return _SC_CACHE["k"]


def _tc_prep_body(cnt_ref, x_ref, dinv_ref, xp_ref):
    deg = cnt_ref[0:1, :] + cnt_ref[1:2, :] + 1.0
    dinv_row = lax.rsqrt(deg)
    eye = (lax.broadcasted_iota(jnp.int32, (128, 128), 0)
           == lax.broadcasted_iota(jnp.int32, (128, 128), 1)).astype(jnp.float32)
    blocks = [
        jnp.sum(eye * dinv_row[:, i * 128:(i + 1) * 128], axis=1,
                keepdims=True)
        for i in range(_NP // 128)
    ]
    dinv = jnp.concatenate(blocks, axis=0)
    dinv_ref[...] = dinv
    xp_ref[...] = x_ref[...] * dinv


_tc_prep = pl.pallas_call(
    _tc_prep_body,
    out_shape=(
        jax.ShapeDtypeStruct((_NP, 1), jnp.float32),
        jax.ShapeDtypeStruct((_NP, _D), jnp.float32),
    ),
)


def _tc_layer_body(p_ref, xp_ref, dinv_ref, w_ref, b_ref, g_ref, bt_ref,
                   o_ref):
    dinv = dinv_ref[...]
    t = (p_ref[0] + p_ref[1] + xp_ref[...]) * dinv
    y = jnp.dot(t, w_ref[...], preferred_element_type=jnp.float32) + b_ref[...]
    yr = y[:_N]
    m = jnp.mean(yr, axis=0, keepdims=True)
    v = jnp.mean((yr - m) * (yr - m), axis=0, keepdims=True)
    h = (y - m) * lax.rsqrt(v + _EPS) * g_ref[...] + bt_ref[...]
    o_ref[...] = jnp.maximum(h, 0.0) * dinv


_tc_layer = pl.pallas_call(
    _tc_layer_body,
    out_shape=jax.ShapeDtypeStruct((_NP, _D), jnp.float32),
)


def _tc_final_body(p_ref, xp_ref, dinv_ref, w_ref, b_ref, x_ref, o_ref):
    t = (p_ref[0] + p_ref[1] + xp_ref[...]) * dinv_ref[...]
    y = jnp.dot(t, w_ref[...], preferred_element_type=jnp.float32)
    o_ref[...] = y + b_ref[...] + x_ref[...]


_tc_final = pl.pallas_call(
    _tc_final_body,
    out_shape=jax.ShapeDtypeStruct((_NP, _D), jnp.float32),
)


def kernel(x, edge_index, W1, b1, g1, bt1, W2, b2, g2, bt2, W3, b3):
    sc_counts, sc_spmm = _sc_kernels()
    src = edge_index[0].astype(jnp.int32)
    dst = edge_index[1].astype(jnp.int32)
    pad = jnp.full((_EPAD - _E,), _N, jnp.int32)
    srcp = jnp.concatenate([src, pad])
    dstp = jnp.concatenate([dst, pad])
    xpd = jnp.pad(x, ((0, _NP - _N), (0, 0)))

    cnt = sc_counts(dstp)
    dinv, xp1 = _tc_prep(cnt, xpd)
    p1 = sc_spmm(xp1, srcp, dstp)
    xp2 = _tc_layer(p1, xp1, dinv, W1, b1.reshape(1, _D),
                    g1.reshape(1, _D), bt1.reshape(1, _D))
    p2 = sc_spmm(xp2, srcp, dstp)
    xp3 = _tc_layer(p2, xp2, dinv, W2, b2.reshape(1, _D),
                    g2.reshape(1, _D), bt2.reshape(1, _D))
    p3 = sc_spmm(xp3, srcp, dstp)
    out = _tc_final(p3, xp3, dinv, W3, b3.reshape(1, _D), xpd)
    return out[:_N]


# speedup vs baseline: 1.0361x; 1.0361x over previous
import functools

import jax
import jax.numpy as jnp
from jax import lax
from jax.experimental import pallas as pl
from jax.experimental.pallas import tpu as pltpu, tpu_sc as plsc

_N = 10000
_D = 128
_E = 320000
_EPS = 1e-5

_NP = 10240
_EPAD = 327680
_EPT = _EPAD // 32
_CH = 128
_NCHUNK = 2 * _EPT // _CH
_BC = 4
_CH0 = 120
_CH1 = _NCHUNK - _CH0
_RPT = _NP // 16
_CHC = 128
_NCHUNKC = _EPT // _CHC

_SC_CACHE = {}


def _sc_kernels():
    if "k" in _SC_CACHE:
        return _SC_CACHE["k"]
    mesh = plsc.VectorSubcoreMesh(core_axis_name="c", subcore_axis_name="s")

    @functools.partial(
        pl.kernel,
        mesh=mesh,
        out_type=jax.ShapeDtypeStruct((2, _NP), jnp.float32),
        scratch_types=[
            pltpu.VMEM((_EPT,), jnp.int32),
            pltpu.VMEM((_CHC,), jnp.int32),
            pltpu.VMEM((_CHC,), jnp.float32),
            pltpu.VMEM((_RPT,), jnp.float32),
            pltpu.VMEM_SHARED((_NP,), jnp.float32),
        ],
    )
    def sc_counts(dst_hbm, out_hbm, dst_1d, dst_c, ones_v, zero_v, acc_sh):
        cid = lax.axis_index("c")
        sid = lax.axis_index("s")
        wid = sid * 2 + cid
        for k in range(_CHC // 16):
            ones_v[pl.ds(k * 16, 16)] = jnp.ones((16,), jnp.float32)

        def zbody(k, carry):
            zero_v[pl.ds(k * 16, 16)] = jnp.zeros((16,), jnp.float32)
            return carry

        lax.fori_loop(0, _RPT // 16, zbody, 0)
        pltpu.sync_copy(zero_v, acc_sh.at[pl.ds(sid * _RPT, _RPT)])
        pltpu.sync_copy(dst_hbm.at[pl.ds(wid * _EPT, _EPT)], dst_1d)
        plsc.subcore_barrier()

        def body(j, carry):
            for k in range(_CHC // 16):
                dst_c[pl.ds(k * 16, 16)] = dst_1d[pl.ds(j * _CHC + k * 16, 16)]
            pltpu.sync_copy(ones_v, acc_sh.at[dst_c], add=True)
            return carry

        lax.fori_loop(0, _NCHUNKC, body, 0)
        plsc.subcore_barrier()
        pltpu.sync_copy(acc_sh.at[pl.ds(sid * _RPT, _RPT)],
                        out_hbm.at[cid, pl.ds(sid * _RPT, _RPT)])

    @functools.partial(
        pl.kernel,
        mesh=mesh,
        out_type=jax.ShapeDtypeStruct((2, _NP, _D), jnp.float32),
        scratch_types=(
            [pltpu.VMEM((_BC * _CH,), jnp.int32) for _ in range(4)]
            + [pltpu.VMEM((_CH,), jnp.int32) for _ in range(2)]
            + [pltpu.VMEM((_CH, _D), jnp.float32) for _ in range(2)]
            + [pltpu.VMEM_SHARED((_NP, _D), jnp.float32)]
            + [pltpu.SemaphoreType.DMA for _ in range(6)]
        ),
    )
    def sc_spmm(xp_hbm, src_hbm, dst_hbm, out_hbm, *scratch):
        sblk = scratch[0:2]
        dblk = scratch[2:4]
        dst_v = scratch[4:6]
        rows = scratch[6:8]
        acc_sh = scratch[8]
        sems = scratch[9:11]
        sem_s = scratch[11:13]
        sem_d = scratch[13:15]
        cid = lax.axis_index("c")
        sid = lax.axis_index("s")

        def zbody(r, carry):
            for k in range(_D // 16):
                rows[0][r, pl.ds(k * 16, 16)] = jnp.zeros((16,), jnp.float32)
            return carry

        lax.fori_loop(0, _CH, zbody, 0)
        for k in range(_RPT // _CH):
            pltpu.sync_copy(rows[0],
                            acc_sh.at[pl.ds(sid * _RPT + k * _CH, _CH), :])
        plsc.subcore_barrier()

        def run_edges(base_chunk, nblocks):
            def load_blk(b, q):
                base = (base_chunk + b * _BC) * _CH
                pltpu.async_copy(src_hbm.at[pl.ds(base, _BC * _CH)],
                                 sblk[q], sem_s[q])
                pltpu.async_copy(dst_hbm.at[pl.ds(base, _BC * _CH)],
                                 dblk[q], sem_d[q])

            def wait_blk(q):
                pltpu.make_async_copy(src_hbm.at[pl.ds(0, _BC * _CH)],
                                      sblk[q], sem_s[q]).wait()
                pltpu.make_async_copy(dst_hbm.at[pl.ds(0, _BC * _CH)],
                                      dblk[q], sem_d[q]).wait()

            def do_block(q):
                def fetch(i, p):
                    for k in range(_CH // 16):
                        dst_v[p][pl.ds(k * 16, 16)] = (
                            dblk[q][pl.ds(i * _CH + k * 16, 16)])
                    pltpu.async_copy(
                        xp_hbm.at[sblk[q].at[pl.ds(i * _CH, _CH)]],
                        rows[p], sems[p])

                def drain(p):
                    pltpu.make_async_copy(
                        xp_hbm.at[sblk[q].at[pl.ds(0, _CH)]],
                        rows[p], sems[p]).wait()
                    pltpu.sync_copy(rows[p], acc_sh.at[dst_v[p]], add=True)

                fetch(0, 0)
                for m in range(_BC // 2):
                    fetch(2 * m + 1, 1)
                    drain(0)
                    if m < _BC // 2 - 1:
                        fetch(2 * m + 2, 0)
                    drain(1)

            load_blk(0, 0)

            def body(m, carry):
                load_blk(2 * m + 1, 1)
                wait_blk(0)
                do_block(0)

                @pl.when(m < nblocks // 2 - 1)
                def _():
                    load_blk(2 * m + 2, 0)

                wait_blk(1)
                do_block(1)
                return carry

            lax.fori_loop(0, nblocks // 2, body, 0)

        @pl.when(cid == 0)
        def _():
            run_edges(sid * _NCHUNK, _CH0 // _BC)

        @pl.when(cid == 1)
        def _():
            run_edges(sid * _NCHUNK + _CH0, _CH1 // _BC)
        plsc.subcore_barrier()
        pltpu.sync_copy(acc_sh.at[pl.ds(sid * _RPT, _RPT), :],
                        out_hbm.at[cid, pl.ds(sid * _RPT, _RPT), :])

    _SC_CACHE["k"] = (sc_counts, sc_spmm)
    return _SC_CACHE["k"]


def _tc_prep_body(cnt_ref, x_ref, dinv_ref, xp_ref):
    deg = cnt_ref[0:1, :] + cnt_ref[1:2, :] + 1.0
    dinv_row = lax.rsqrt(deg)
    eye = (lax.broadcasted_iota(jnp.int32, (128, 128), 0)
           == lax.broadcasted_iota(jnp.int32, (128, 128), 1)).astype(jnp.float32)
    blocks = [
        jnp.sum(eye * dinv_row[:, i * 128:(i + 1) * 128], axis=1,
                keepdims=True)
        for i in range(_NP // 128)
    ]
    dinv = jnp.concatenate(blocks, axis=0)
    dinv_ref[...] = dinv
    xp_ref[...] = x_ref[...] * dinv


_tc_prep = pl.pallas_call(
    _tc_prep_body,
    out_shape=(
        jax.ShapeDtypeStruct((_NP, 1), jnp.float32),
        jax.ShapeDtypeStruct((_NP, _D), jnp.float32),
    ),
)


def _tc_layer_body(p_ref, xp_ref, dinv_ref, w_ref, b_ref, g_ref, bt_ref,
                   o_ref):
    dinv = dinv_ref[...]
    t = (p_ref[0] + p_ref[1] + xp_ref[...]) * dinv
    y = jnp.dot(t, w_ref[...], preferred_element_type=jnp.float32) + b_ref[...]
    yr = y[:_N]
    m = jnp.mean(yr, axis=0, keepdims=True)
    v = jnp.mean((yr - m) * (yr - m), axis=0, keepdims=True)
    h = (y - m) * lax.rsqrt(v + _EPS) * g_ref[...] + bt_ref[...]
    o_ref[...] = jnp.maximum(h, 0.0) * dinv


_tc_layer = pl.pallas_call(
    _tc_layer_body,
    out_shape=jax.ShapeDtypeStruct((_NP, _D), jnp.float32),
)


def _tc_final_body(p_ref, xp_ref, dinv_ref, w_ref, b_ref, x_ref, o_ref):
    t = (p_ref[0] + p_ref[1] + xp_ref[...]) * dinv_ref[...]
    y = jnp.dot(t, w_ref[...], preferred_element_type=jnp.float32)
    o_ref[...] = y + b_ref[...] + x_ref[...]


_tc_final = pl.pallas_call(
    _tc_final_body,
    out_shape=jax.ShapeDtypeStruct((_NP, _D), jnp.float32),
)


def kernel(x, edge_index, W1, b1, g1, bt1, W2, b2, g2, bt2, W3, b3):
    sc_counts, sc_spmm = _sc_kernels()
    src = edge_index[0].astype(jnp.int32)
    dst = edge_index[1].astype(jnp.int32)
    pad = jnp.full((_EPAD - _E,), _N, jnp.int32)
    srcp = jnp.concatenate([src, pad])
    dstp = jnp.concatenate([dst, pad])
    xpd = jnp.pad(x, ((0, _NP - _N), (0, 0)))

    cnt = sc_counts(dstp)
    dinv, xp1 = _tc_prep(cnt, xpd)
    p1 = sc_spmm(xp1, srcp, dstp)
    xp2 = _tc_layer(p1, xp1, dinv, W1, b1.reshape(1, _D),
                    g1.reshape(1, _D), bt1.reshape(1, _D))
    p2 = sc_spmm(xp2, srcp, dstp)
    xp3 = _tc_layer(p2, xp2, dinv, W2, b2.reshape(1, _D),
                    g2.reshape(1, _D), bt2.reshape(1, _D))
    p3 = sc_spmm(xp3, srcp, dstp)
    out = _tc_final(p3, xp3, dinv, W3, b3.reshape(1, _D), xpd)
    return out[:_N]


# speedup vs baseline: 1.0426x; 1.0062x over previous
import functools

import jax
import jax.numpy as jnp
from jax import lax
from jax.experimental import pallas as pl
from jax.experimental.pallas import tpu as pltpu, tpu_sc as plsc

_N = 10000
_D = 128
_E = 320000
_EPS = 1e-5

_NP = 10240
_EPAD = 327680
_EPT = _EPAD // 32
_CH = 128
_NCHUNK = 2 * _EPT // _CH
_CH0 = 118
_CH1 = _NCHUNK - _CH0
_RPT = _NP // 16
_CHC = 128
_NCHUNKC = _EPT // _CHC

_SC_CACHE = {}


def _sc_kernels():
    if "k" in _SC_CACHE:
        return _SC_CACHE["k"]
    mesh = plsc.VectorSubcoreMesh(core_axis_name="c", subcore_axis_name="s")

    @functools.partial(
        pl.kernel,
        mesh=mesh,
        out_type=jax.ShapeDtypeStruct((2, _NP), jnp.float32),
        scratch_types=[
            pltpu.VMEM((_EPT,), jnp.int32),
            pltpu.VMEM((_CHC,), jnp.int32),
            pltpu.VMEM((_CHC,), jnp.float32),
            pltpu.VMEM((_RPT,), jnp.float32),
            pltpu.VMEM_SHARED((_NP,), jnp.float32),
        ],
    )
    def sc_counts(dst_hbm, out_hbm, dst_1d, dst_c, ones_v, zero_v, acc_sh):
        cid = lax.axis_index("c")
        sid = lax.axis_index("s")
        wid = sid * 2 + cid
        for k in range(_CHC // 16):
            ones_v[pl.ds(k * 16, 16)] = jnp.ones((16,), jnp.float32)

        def zbody(k, carry):
            zero_v[pl.ds(k * 16, 16)] = jnp.zeros((16,), jnp.float32)
            return carry

        lax.fori_loop(0, _RPT // 16, zbody, 0)
        pltpu.sync_copy(zero_v, acc_sh.at[pl.ds(sid * _RPT, _RPT)])
        pltpu.sync_copy(dst_hbm.at[pl.ds(wid * _EPT, _EPT)], dst_1d)
        plsc.subcore_barrier()

        def body(j, carry):
            for k in range(_CHC // 16):
                dst_c[pl.ds(k * 16, 16)] = dst_1d[pl.ds(j * _CHC + k * 16, 16)]
            pltpu.sync_copy(ones_v, acc_sh.at[dst_c], add=True)
            return carry

        lax.fori_loop(0, _NCHUNKC, body, 0)
        plsc.subcore_barrier()
        pltpu.sync_copy(acc_sh.at[pl.ds(sid * _RPT, _RPT)],
                        out_hbm.at[cid, pl.ds(sid * _RPT, _RPT)])

    @functools.partial(
        pl.kernel,
        mesh=mesh,
        out_type=jax.ShapeDtypeStruct((2, _NP, _D), jnp.float32),
        scratch_types=(
            [pltpu.VMEM((_CH,), jnp.int32) for _ in range(4)]
            + [pltpu.VMEM((_CH, _D), jnp.float32) for _ in range(2)]
            + [pltpu.VMEM_SHARED((_NP, _D), jnp.float32)]
            + [pltpu.SemaphoreType.DMA for _ in range(2)]
        ),
    )
    def sc_spmm(xp_hbm, src_hbm, dst_hbm, out_hbm, *scratch):
        src_v = scratch[0:2]
        dst_v = scratch[2:4]
        rows = scratch[4:6]
        acc_sh = scratch[6]
        sems = scratch[7:9]
        cid = lax.axis_index("c")
        sid = lax.axis_index("s")

        def zbody(r, carry):
            for k in range(_D // 16):
                rows[0][r, pl.ds(k * 16, 16)] = jnp.zeros((16,), jnp.float32)
            return carry

        lax.fori_loop(0, _CH, zbody, 0)
        for k in range(_RPT // _CH):
            pltpu.sync_copy(rows[0],
                            acc_sh.at[pl.ds(sid * _RPT + k * _CH, _CH), :])
        plsc.subcore_barrier()

        def run_edges(base_chunk, nchunks):
            def fetch(j, p):
                base = (base_chunk + j) * _CH
                pltpu.sync_copy(src_hbm.at[pl.ds(base, _CH)], src_v[p])
                pltpu.sync_copy(dst_hbm.at[pl.ds(base, _CH)], dst_v[p])
                pltpu.async_copy(xp_hbm.at[src_v[p]], rows[p], sems[p])

            def drain(p):
                pltpu.make_async_copy(xp_hbm.at[src_v[p]], rows[p],
                                      sems[p]).wait()
                pltpu.sync_copy(rows[p], acc_sh.at[dst_v[p]], add=True)

            fetch(0, 0)

            def body(m, carry):
                fetch(2 * m + 1, 1)
                drain(0)

                @pl.when(m < nchunks // 2 - 1)
                def _():
                    fetch(2 * m + 2, 0)

                drain(1)
                return carry

            lax.fori_loop(0, nchunks // 2, body, 0)

        @pl.when(cid == 0)
        def _():
            run_edges(sid * _NCHUNK, _CH0)

        @pl.when(cid == 1)
        def _():
            run_edges(sid * _NCHUNK + _CH0, _CH1)
        plsc.subcore_barrier()
        pltpu.sync_copy(acc_sh.at[pl.ds(sid * _RPT, _RPT), :],
                        out_hbm.at[cid, pl.ds(sid * _RPT, _RPT), :])

    _SC_CACHE["k"] = (sc_counts, sc_spmm)
    return _SC_CACHE["k"]


def _tc_prep_body(cnt_ref, x_ref, dinv_ref, xp_ref):
    deg = cnt_ref[0:1, :] + cnt_ref[1:2, :] + 1.0
    dinv_row = lax.rsqrt(deg)
    eye = (lax.broadcasted_iota(jnp.int32, (128, 128), 0)
           == lax.broadcasted_iota(jnp.int32, (128, 128), 1)).astype(jnp.float32)
    blocks = [
        jnp.sum(eye * dinv_row[:, i * 128:(i + 1) * 128], axis=1,
                keepdims=True)
        for i in range(_NP // 128)
    ]
    dinv = jnp.concatenate(blocks, axis=0)
    dinv_ref[...] = dinv
    xp_ref[...] = x_ref[...] * dinv


_tc_prep = pl.pallas_call(
    _tc_prep_body,
    out_shape=(
        jax.ShapeDtypeStruct((_NP, 1), jnp.float32),
        jax.ShapeDtypeStruct((_NP, _D), jnp.float32),
    ),
)


def _tc_layer_body(p_ref, xp_ref, dinv_ref, w_ref, b_ref, g_ref, bt_ref,
                   o_ref):
    dinv = dinv_ref[...]
    t = (p_ref[0] + p_ref[1] + xp_ref[...]) * dinv
    y = jnp.dot(t, w_ref[...], preferred_element_type=jnp.float32) + b_ref[...]
    yr = y[:_N]
    m = jnp.mean(yr, axis=0, keepdims=True)
    v = jnp.mean((yr - m) * (yr - m), axis=0, keepdims=True)
    h = (y - m) * lax.rsqrt(v + _EPS) * g_ref[...] + bt_ref[...]
    o_ref[...] = jnp.maximum(h, 0.0) * dinv


_tc_layer = pl.pallas_call(
    _tc_layer_body,
    out_shape=jax.ShapeDtypeStruct((_NP, _D), jnp.float32),
)


def _tc_final_body(p_ref, xp_ref, dinv_ref, w_ref, b_ref, x_ref, o_ref):
    t = (p_ref[0] + p_ref[1] + xp_ref[...]) * dinv_ref[...]
    y = jnp.dot(t, w_ref[...], preferred_element_type=jnp.float32)
    o_ref[...] = y + b_ref[...] + x_ref[...]


_tc_final = pl.pallas_call(
    _tc_final_body,
    out_shape=jax.ShapeDtypeStruct((_NP, _D), jnp.float32),
)


def kernel(x, edge_index, W1, b1, g1, bt1, W2, b2, g2, bt2, W3, b3):
    sc_counts, sc_spmm = _sc_kernels()
    src = edge_index[0].astype(jnp.int32)
    dst = edge_index[1].astype(jnp.int32)
    pad = jnp.full((_EPAD - _E,), _N, jnp.int32)
    srcp = jnp.concatenate([src, pad])
    dstp = jnp.concatenate([dst, pad])
    xpd = jnp.pad(x, ((0, _NP - _N), (0, 0)))

    cnt = sc_counts(dstp)
    dinv, xp1 = _tc_prep(cnt, xpd)
    p1 = sc_spmm(xp1, srcp, dstp)
    xp2 = _tc_layer(p1, xp1, dinv, W1, b1.reshape(1, _D),
                    g1.reshape(1, _D), bt1.reshape(1, _D))
    p2 = sc_spmm(xp2, srcp, dstp)
    xp3 = _tc_layer(p2, xp2, dinv, W2, b2.reshape(1, _D),
                    g2.reshape(1, _D), bt2.reshape(1, _D))
    p3 = sc_spmm(xp3, srcp, dstp)
    out = _tc_final(p3, xp3, dinv, W3, b3.reshape(1, _D), xpd)
    return out[:_N]
